# baseline (speedup 1.0000x reference)
import functools

import jax
import jax.numpy as jnp
from jax.experimental import pallas as pl
from jax.experimental.pallas import tpu as pltpu
from jax.experimental.pallas import tpu_sc as plsc

B = 4
S = 8192
D = 768
N = B * S
C = 128
NC = D // C
W = 128
NWIN = N // W
LANES = 16


def _pe_add_sc(x2d, fidx, table_flat):
    mesh = plsc.VectorSubcoreMesh(core_axis_name="c", subcore_axis_name="s")

    @functools.partial(
        pl.kernel,
        out_type=jax.ShapeDtypeStruct((N, D), jnp.float32),
        mesh=mesh,
        scratch_types=[pltpu.VMEM((W, C), jnp.float32)],
    )
    def k(x_hbm, i_hbm, t_hbm, o_hbm, rows_v):
        def body(i_vmem, x_vmem, o_vmem):
            pltpu.sync_copy(t_hbm.at[i_vmem.at[0]], rows_v)

            @pl.loop(0, W)
            def _row(r):
                @pl.loop(0, C, step=LANES)
                def _col(c):
                    slc = (pl.ds(r, 1), pl.ds(c, LANES))
                    o_vmem.at[slc][...] = x_vmem.at[slc][...] + rows_v.at[slc][...]

        pltpu.emit_pipeline(
            body,
            grid=(NWIN, NC),
            in_specs=[
                pl.BlockSpec((1, W), lambda i, j: (i * NC + j, 0)),
                pl.BlockSpec((W, C), lambda i, j: (i, j)),
            ],
            out_specs=[pl.BlockSpec((W, C), lambda i, j: (i, j))],
            core_axis_name=("c", "s"),
            dimension_semantics=(pltpu.PARALLEL, pltpu.PARALLEL),
        )(i_hbm, x_hbm, o_hbm)

    return k(x2d, fidx, table_flat)


def kernel(x, position, pos_table):
    x2d = x.reshape(N, D)
    pos = position.reshape(NWIN, W).astype(jnp.int32)
    fidx = (pos[:, None, :] * NC + jnp.arange(NC, dtype=jnp.int32)[None, :, None])
    fidx = fidx.reshape(NWIN * NC, W)
    table_flat = pos_table.reshape(8192 * NC, C)
    out = _pe_add_sc(x2d, fidx, table_flat)
    return out.reshape(B, S, D)


# speedup vs baseline: 1.1586x; 1.1586x over previous
import functools

import jax
import jax.numpy as jnp
from jax.experimental import pallas as pl
from jax.experimental.pallas import tpu as pltpu
from jax.experimental.pallas import tpu_sc as plsc

B = 4
S = 8192
D = 768
N = B * S
C = 128
NC = D // C
W = 128
NWIN = N // W
LANES = 16


def _pe_add_sc(x2d, fidx, table_flat):
    mesh = plsc.VectorSubcoreMesh(core_axis_name="c", subcore_axis_name="s")

    @functools.partial(
        pl.kernel,
        out_type=jax.ShapeDtypeStruct((N, D), jnp.float32),
        mesh=mesh,
    )
    def k(x_hbm, i_hbm, t_hbm, o_hbm):
        def body(i_vmem, x_vmem, o_vmem):
            pltpu.sync_copy(t_hbm.at[i_vmem.at[0]], o_vmem)

            @pl.loop(0, W)
            def _row(r):
                @pl.loop(0, C, step=LANES)
                def _col(c):
                    slc = (pl.ds(r, 1), pl.ds(c, LANES))
                    plsc.addupdate(o_vmem.at[slc], x_vmem.at[slc][...])

        pltpu.emit_pipeline(
            body,
            grid=(NWIN, NC),
            in_specs=[
                pl.BlockSpec((1, W), lambda i, j: (i * NC + j, 0)),
                pl.BlockSpec((W, C), lambda i, j: (i, j)),
            ],
            out_specs=[pl.BlockSpec((W, C), lambda i, j: (i, j))],
            core_axis_name=("c", "s"),
            dimension_semantics=(pltpu.PARALLEL, pltpu.PARALLEL),
        )(i_hbm, x_hbm, o_hbm)

    return k(x2d, fidx, table_flat)


def kernel(x, position, pos_table):
    x2d = x.reshape(N, D)
    pos = position.reshape(NWIN, W).astype(jnp.int32)
    fidx = (pos[:, None, :] * NC + jnp.arange(NC, dtype=jnp.int32)[None, :, None])
    fidx = fidx.reshape(NWIN * NC, W)
    table_flat = pos_table.reshape(8192 * NC, C)
    out = _pe_add_sc(x2d, fidx, table_flat)
    return out.reshape(B, S, D)


# speedup vs baseline: 1.1588x; 1.0002x over previous
import functools

import jax
import jax.numpy as jnp
from jax.experimental import pallas as pl
from jax.experimental.pallas import tpu as pltpu
from jax.experimental.pallas import tpu_sc as plsc

B = 4
S = 8192
D = 768
N = B * S
C = 128
NC = D // C
W = 128
NWIN = N // W
LANES = 16


def _pe_add_sc(x2d, fidx, table_flat):
    mesh = plsc.VectorSubcoreMesh(core_axis_name="c", subcore_axis_name="s")

    @functools.partial(
        pl.kernel,
        out_type=jax.ShapeDtypeStruct((N, D), jnp.float32),
        mesh=mesh,
    )
    def k(x_hbm, i_hbm, t_hbm, o_hbm):
        def body(i_vmem, x_vmem, o_vmem):
            pltpu.sync_copy(t_hbm.at[i_vmem.at[0]], o_vmem)

            @pl.loop(0, W)
            def _row(r):
                for c in range(0, C, LANES):
                    slc = (pl.ds(r, 1), pl.ds(c, LANES))
                    plsc.addupdate(o_vmem.at[slc], x_vmem.at[slc][...])

        pltpu.emit_pipeline(
            body,
            grid=(NWIN, NC),
            in_specs=[
                pl.BlockSpec((1, W), lambda i, j: (i * NC + j, 0)),
                pl.BlockSpec((W, C), lambda i, j: (i, j)),
            ],
            out_specs=[pl.BlockSpec((W, C), lambda i, j: (i, j))],
            core_axis_name=("c", "s"),
            dimension_semantics=(pltpu.PARALLEL, pltpu.PARALLEL),
        )(i_hbm, x_hbm, o_hbm)

    return k(x2d, fidx, table_flat)


def kernel(x, position, pos_table):
    x2d = x.reshape(N, D)
    pos = position.reshape(NWIN, W).astype(jnp.int32)
    fidx = (pos[:, None, :] * NC + jnp.arange(NC, dtype=jnp.int32)[None, :, None])
    fidx = fidx.reshape(NWIN * NC, W)
    table_flat = pos_table.reshape(8192 * NC, C)
    out = _pe_add_sc(x2d, fidx, table_flat)
    return out.reshape(B, S, D)


# speedup vs baseline: 1.7695x; 1.5270x over previous
import functools

import jax
import jax.numpy as jnp
from jax.experimental import pallas as pl
from jax.experimental.pallas import tpu as pltpu
from jax.experimental.pallas import tpu_sc as plsc

B = 4
S = 8192
D = 768
N = B * S
C = 128
NC = D // C
W = 128
NWIN = N // W

TC_ROWS = 1024


def _gather_sc(fidx, table_flat):
    mesh = plsc.VectorSubcoreMesh(core_axis_name="c", subcore_axis_name="s")

    @functools.partial(
        pl.kernel,
        out_type=jax.ShapeDtypeStruct((N, D), jnp.float32),
        mesh=mesh,
    )
    def k(i_hbm, t_hbm, o_hbm):
        def body(i_vmem, o_vmem):
            pltpu.sync_copy(t_hbm.at[i_vmem.at[0]], o_vmem)

        pltpu.emit_pipeline(
            body,
            grid=(NWIN, NC),
            in_specs=[pl.BlockSpec((1, W), lambda i, j: (i * NC + j, 0))],
            out_specs=[pl.BlockSpec((W, C), lambda i, j: (i, j))],
            core_axis_name=("c", "s"),
            dimension_semantics=(pltpu.PARALLEL, pltpu.PARALLEL),
        )(i_hbm, o_hbm)

    return k(fidx, table_flat)


def _add_tc(x2d, pe2d):
    def body(x_ref, pe_ref, o_ref):
        o_ref[...] = x_ref[...] + pe_ref[...]

    return pl.pallas_call(
        body,
        out_shape=jax.ShapeDtypeStruct((N, D), jnp.float32),
        grid=(N // TC_ROWS,),
        in_specs=[
            pl.BlockSpec((TC_ROWS, D), lambda i: (i, 0)),
            pl.BlockSpec((TC_ROWS, D), lambda i: (i, 0)),
        ],
        out_specs=pl.BlockSpec((TC_ROWS, D), lambda i: (i, 0)),
    )(x2d, pe2d)


def kernel(x, position, pos_table):
    x2d = x.reshape(N, D)
    pos = position.reshape(NWIN, W).astype(jnp.int32)
    fidx = (pos[:, None, :] * NC + jnp.arange(NC, dtype=jnp.int32)[None, :, None])
    fidx = fidx.reshape(NWIN * NC, W)
    table_flat = pos_table.reshape(8192 * NC, C)
    pe2d = _gather_sc(fidx, table_flat)
    out = _add_tc(x2d, pe2d)
    return out.reshape(B, S, D)
